# baseline (speedup 1.0000x reference)
import functools

import jax
import jax.numpy as jnp
from jax import lax
from jax.experimental import pallas as pl
import jax.experimental.pallas.tpu as pltpu
from jax.experimental.pallas import tpu_sc as plsc

N = 50000
NP = 50048
E = 800000
F_IN = 50
HID = 256
OUT = 121

NC = 2
NS = 16
CHUNK = 128
K = 1
NSLOT = 6
NCHUNK = 6336
E_PAD = NCHUNK * CHUNK
NITER_AGG = NCHUNK // NS // K
NITER_DEG = NCHUNK // (NC * NS) // K
ROWS_PER_TILE = NP // NS
RB = 1088
GRID = NP // RB

_mesh = plsc.VectorSubcoreMesh(core_axis_name="c", subcore_axis_name="s")
_sc_params = pltpu.CompilerParams(use_tc_tiling_on_sc=False)


def _deg_body(edge_hbm, ones_hbm, out_hbm, acc, idxb, onesb, sem_s):
  c = lax.axis_index("c")
  t = lax.axis_index("s")
  w = c * NS + t
  pltpu.sync_copy(ones_hbm.at[pl.ds(t * ROWS_PER_TILE, ROWS_PER_TILE)],
                  acc.at[pl.ds(t * ROWS_PER_TILE, ROWS_PER_TILE)])
  pltpu.sync_copy(ones_hbm.at[pl.ds(0, CHUNK)], onesb)
  plsc.subcore_barrier()

  start = w * (NITER_DEG * K)

  def stage(i, slot):
    pltpu.sync_copy(edge_hbm.at[1, pl.ds(start + i * K, K), :], idxb.at[slot])

  def scat(slot):
    for j in range(K):
      pltpu.async_copy(onesb, acc.at[idxb.at[slot, j]], sem_s, add=True)

  def wait_scat(slot):
    for j in range(K):
      pltpu.make_async_copy(onesb, acc.at[idxb.at[slot, j]], sem_s).wait()

  stage(0, 0)

  def body(i, carry):
    s0, s1 = carry

    @pl.when(i >= NSLOT - 1)
    def _():
      wait_scat(s1)

    @pl.when(i + 1 < NITER_DEG)
    def _():
      stage(i + 1, s1)

    scat(s0)
    return s1, jnp.where(s1 == NSLOT - 1, 0, s1 + 1)

  lax.fori_loop(0, NITER_DEG, body,
                (jnp.int32(0), jnp.int32(1)))
  for d in range(1, NSLOT):
    wait_scat((NITER_DEG - d) % NSLOT)
  plsc.subcore_barrier()
  pltpu.sync_copy(acc.at[pl.ds(t * ROWS_PER_TILE, ROWS_PER_TILE)],
                  out_hbm.at[c, pl.ds(t * ROWS_PER_TILE, ROWS_PER_TILE)])


_deg_call = pl.kernel(
    _deg_body,
    out_type=jax.ShapeDtypeStruct((NC, NP, 16), jnp.float32),
    mesh=_mesh,
    compiler_params=_sc_params,
    scratch_types=[
        pltpu.VMEM_SHARED((NP, 16), jnp.float32),
        pltpu.VMEM((NSLOT, K, CHUNK), jnp.int32),
        pltpu.VMEM((CHUNK, 16), jnp.float32),
        pltpu.SemaphoreType.DMA,
    ],
)


def _agg_body(nchunks, table_hbm, edge_hbm, out_hbm,
              acc, idxb, rows, sem_g, sem_s):
  c = lax.axis_index("c")
  t = lax.axis_index("s")
  start = t * (NITER_AGG * K)

  for k in range(nchunks // NC):
    p = c + NC * k
    pbase = p * NP
    tview = table_hbm.at[pl.ds(pbase, NP)]
    pltpu.sync_copy(
        table_hbm.at[pl.ds(pbase + t * ROWS_PER_TILE, ROWS_PER_TILE)],
        acc.at[pl.ds(t * ROWS_PER_TILE, ROWS_PER_TILE)])
    plsc.subcore_barrier()

    def stage(i, slot):
      pltpu.sync_copy(edge_hbm.at[:, pl.ds(start + i * K, K), :], idxb.at[slot])

    def gath(slot):
      for j in range(K):
        pltpu.async_copy(tview.at[idxb.at[slot, 0, j]], rows.at[slot, j], sem_g)

    def wait_gath(slot):
      for j in range(K):
        pltpu.make_async_copy(tview.at[idxb.at[slot, 0, j]],
                              rows.at[slot, j], sem_g).wait()

    def scat(slot):
      for j in range(K):
        pltpu.async_copy(rows.at[slot, j], acc.at[idxb.at[slot, 1, j]],
                         sem_s, add=True)

    def wait_scat(slot):
      for j in range(K):
        pltpu.make_async_copy(rows.at[slot, j],
                              acc.at[idxb.at[slot, 1, j]], sem_s).wait()

    stage(0, 0)
    gath(0)

    def body(i, carry):
      s0, s1 = carry

      @pl.when(i >= NSLOT - 1)
      def _():
        wait_scat(s1)

      @pl.when(i + 1 < NITER_AGG)
      def _():
        stage(i + 1, s1)
        gath(s1)

      wait_gath(s0)
      scat(s0)
      return s1, jnp.where(s1 == NSLOT - 1, 0, s1 + 1)

    lax.fori_loop(0, NITER_AGG, body,
                  (jnp.int32(0), jnp.int32(1)))
    for d in range(1, NSLOT):
      wait_scat((NITER_AGG - d) % NSLOT)
    plsc.subcore_barrier()
    pltpu.sync_copy(
        acc.at[pl.ds(t * ROWS_PER_TILE, ROWS_PER_TILE)],
        out_hbm.at[pl.ds(pbase + t * ROWS_PER_TILE, ROWS_PER_TILE)])
    plsc.subcore_barrier()


def _make_agg(nchunks):
  return pl.kernel(
      functools.partial(_agg_body, nchunks),
      out_type=jax.ShapeDtypeStruct((nchunks * NP, 32), jnp.float32),
      mesh=_mesh,
      compiler_params=_sc_params,
      scratch_types=[
          pltpu.VMEM_SHARED((NP, 32), jnp.float32),
          pltpu.VMEM((NSLOT, 2, K, CHUNK), jnp.int32),
          pltpu.VMEM((NSLOT, K, CHUNK, 32), jnp.float32),
          pltpu.SemaphoreType.DMA,
          pltpu.SemaphoreType.DMA,
      ],
  )


_agg2_call = _make_agg(2)
_agg4_call = _make_agg(4)


def _prep_body(degp_ref, x_ref, xp_ref, dinv_ref):
  d = degp_ref[0, :, 0:1] + degp_ref[1, :, 0:1] - 1.0
  d = jnp.maximum(d, 1.0)
  dinv = lax.rsqrt(d)
  dinv_ref[...] = jnp.broadcast_to(dinv, (RB, 16))
  xv = x_ref[...] * dinv
  xp_ref[0] = xv[:, :32]
  xp_ref[1] = jnp.concatenate(
      [xv[:, 32:F_IN], jnp.zeros((RB, 64 - F_IN), jnp.float32)], axis=1)


def _mid_body(agg1_ref, dinv_ref, w1_ref, w2_ref, b1_ref, zp_ref):
  dinv = dinv_ref[:, 0:1]
  a = jnp.concatenate([agg1_ref[0], agg1_ref[1]], axis=1) * dinv
  h = lax.dot_general(a, w1_ref[...], (((1,), (0,)), ((), ())),
                      precision=lax.Precision.HIGHEST,
                      preferred_element_type=jnp.float32) + b1_ref[...]
  h = jnp.where(h >= 0, h, 0.01 * h)
  z = lax.dot_general(h, w2_ref[...], (((1,), (0,)), ((), ())),
                      precision=lax.Precision.HIGHEST,
                      preferred_element_type=jnp.float32) * dinv
  for p in range(4):
    zp_ref[p] = z[:, 32 * p:32 * (p + 1)]


def _final_body(agg2_ref, dinv_ref, b2_ref, out_ref):
  dinv = dinv_ref[:, 0:1]
  z = jnp.concatenate([agg2_ref[p] for p in range(4)], axis=1)
  z = z * dinv + b2_ref[...]
  lane = lax.broadcasted_iota(jnp.int32, (RB, 128), 1)
  zm = jnp.where(lane >= OUT, -jnp.inf, z)
  m = jnp.max(zm, axis=1, keepdims=True)
  lse = jnp.log(jnp.sum(jnp.exp(zm - m), axis=1, keepdims=True)) + m
  out_ref[...] = (z - lse)[:, :OUT]


def kernel(x, edge_index, W1, b1, W2, b2):
  ones16 = jnp.ones((NP, 16), jnp.float32)
  xpad = jnp.zeros((NP, F_IN), jnp.float32).at[:N].set(x)

  padrow = N + jnp.arange(E_PAD - E, dtype=jnp.int32) % (NP - N)
  edge3 = jnp.concatenate(
      [edge_index, jnp.broadcast_to(padrow, (2, E_PAD - E))],
      axis=1).reshape(2, NCHUNK, CHUNK)

  degp = _deg_call(edge3, ones16)

  xp, dinv2d = pl.pallas_call(
      _prep_body,
      grid=(GRID,),
      in_specs=[
          pl.BlockSpec((2, RB, 16), lambda i: (0, i, 0)),
          pl.BlockSpec((RB, F_IN), lambda i: (i, 0)),
      ],
      out_specs=[
          pl.BlockSpec((2, RB, 32), lambda i: (0, i, 0)),
          pl.BlockSpec((RB, 16), lambda i: (i, 0)),
      ],
      out_shape=[
          jax.ShapeDtypeStruct((2, NP, 32), jnp.float32),
          jax.ShapeDtypeStruct((NP, 16), jnp.float32),
      ],
  )(degp, xpad)

  agg1 = _agg2_call(xp.reshape(2 * NP, 32), edge3)

  w1p = jnp.zeros((64, HID), jnp.float32).at[:F_IN].set(W1)
  w2p = jnp.zeros((HID, 128), jnp.float32).at[:, :OUT].set(W2)
  b2p = jnp.zeros((1, 128), jnp.float32).at[0, :OUT].set(b2)

  zp = pl.pallas_call(
      _mid_body,
      grid=(GRID,),
      in_specs=[
          pl.BlockSpec((2, RB, 32), lambda i: (0, i, 0)),
          pl.BlockSpec((RB, 16), lambda i: (i, 0)),
          pl.BlockSpec((64, HID), lambda i: (0, 0)),
          pl.BlockSpec((HID, 128), lambda i: (0, 0)),
          pl.BlockSpec((1, HID), lambda i: (0, 0)),
      ],
      out_specs=pl.BlockSpec((4, RB, 32), lambda i: (0, i, 0)),
      out_shape=jax.ShapeDtypeStruct((4, NP, 32), jnp.float32),
  )(agg1.reshape(2, NP, 32), dinv2d, w1p, w2p, b1.reshape(1, HID))

  agg2 = _agg4_call(zp.reshape(4 * NP, 32), edge3)

  out = pl.pallas_call(
      _final_body,
      grid=(GRID,),
      in_specs=[
          pl.BlockSpec((4, RB, 32), lambda i: (0, i, 0)),
          pl.BlockSpec((RB, 16), lambda i: (i, 0)),
          pl.BlockSpec((1, 128), lambda i: (0, 0)),
      ],
      out_specs=pl.BlockSpec((RB, OUT), lambda i: (i, 0)),
      out_shape=jax.ShapeDtypeStruct((N, OUT), jnp.float32),
  )(agg2.reshape(4, NP, 32), dinv2d, b2p)

  return out


# speedup vs baseline: 1.2798x; 1.2798x over previous
import functools

import jax
import jax.numpy as jnp
from jax import lax
from jax.experimental import pallas as pl
import jax.experimental.pallas.tpu as pltpu
from jax.experimental.pallas import tpu_sc as plsc

N = 50000
NP = 50048
E = 800000
F_IN = 50
HID = 256
OUT = 121

NC = 2
NS = 16
CHUNK = 128
K = 2
NSLOT = 3
NCHUNK = 6336
E_PAD = NCHUNK * CHUNK
NITER_AGG = NCHUNK // NS // K
NITER_DEG = NCHUNK // (NC * NS) // K
ROWS_PER_TILE = NP // NS
RB = 1088
GRID = NP // RB

_mesh = plsc.VectorSubcoreMesh(core_axis_name="c", subcore_axis_name="s")
_sc_params = pltpu.CompilerParams(use_tc_tiling_on_sc=False)


def _deg_body(edge_hbm, ones_hbm, out_hbm, acc, idxb, onesb, sem_s):
  c = lax.axis_index("c")
  t = lax.axis_index("s")
  w = c * NS + t
  pltpu.sync_copy(ones_hbm.at[pl.ds(t * ROWS_PER_TILE, ROWS_PER_TILE)],
                  acc.at[pl.ds(t * ROWS_PER_TILE, ROWS_PER_TILE)])
  pltpu.sync_copy(ones_hbm.at[pl.ds(0, CHUNK)], onesb)
  plsc.subcore_barrier()

  start = w * (NITER_DEG * K)

  def stage(i, slot):
    pltpu.sync_copy(edge_hbm.at[1, pl.ds(start + i * K, K), :], idxb.at[slot])

  def scat(slot):
    for j in range(K):
      pltpu.async_copy(onesb, acc.at[idxb.at[slot, j]], sem_s, add=True)

  def wait_scat(slot):
    for j in range(K):
      pltpu.make_async_copy(onesb, acc.at[idxb.at[slot, j]], sem_s).wait()

  stage(0, 0)

  def body(i, carry):
    s0, s1 = carry

    @pl.when(i >= NSLOT - 1)
    def _():
      wait_scat(s1)

    @pl.when(i + 1 < NITER_DEG)
    def _():
      stage(i + 1, s1)

    scat(s0)
    return s1, jnp.where(s1 == NSLOT - 1, 0, s1 + 1)

  lax.fori_loop(0, NITER_DEG, body,
                (jnp.int32(0), jnp.int32(1)))
  for d in range(1, NSLOT):
    wait_scat((NITER_DEG - d) % NSLOT)
  plsc.subcore_barrier()
  pltpu.sync_copy(acc.at[pl.ds(t * ROWS_PER_TILE, ROWS_PER_TILE)],
                  out_hbm.at[c, pl.ds(t * ROWS_PER_TILE, ROWS_PER_TILE)])


_deg_call = pl.kernel(
    _deg_body,
    out_type=jax.ShapeDtypeStruct((NC, NP, 8), jnp.float32),
    mesh=_mesh,
    compiler_params=_sc_params,
    scratch_types=[
        pltpu.VMEM_SHARED((NP, 8), jnp.float32),
        pltpu.VMEM((NSLOT, K, CHUNK), jnp.int32),
        pltpu.VMEM((CHUNK, 8), jnp.float32),
        pltpu.SemaphoreType.DMA,
    ],
)


def _agg_body(nchunks, table_hbm, edge_hbm, out_hbm,
              acc, idxb, rows, sem_g, sem_s):
  c = lax.axis_index("c")
  t = lax.axis_index("s")
  start = t * (NITER_AGG * K)

  for k in range(nchunks // NC):
    p = c + NC * k
    pbase = p * NP
    tview = table_hbm.at[pl.ds(pbase, NP)]

    def stage(i, slot):
      pltpu.sync_copy(edge_hbm.at[:, pl.ds(start + i * K, K), :], idxb.at[slot])

    def gath(slot):
      for j in range(K):
        pltpu.async_copy(tview.at[idxb.at[slot, 0, j]], rows.at[slot, j], sem_g)

    def wait_gath(slot):
      for j in range(K):
        pltpu.make_async_copy(tview.at[idxb.at[slot, 0, j]],
                              rows.at[slot, j], sem_g).wait()

    def scat(slot):
      for j in range(K):
        pltpu.async_copy(rows.at[slot, j], acc.at[idxb.at[slot, 1, j]],
                         sem_s, add=True)

    def wait_scat(slot):
      for j in range(K):
        pltpu.make_async_copy(rows.at[slot, j],
                              acc.at[idxb.at[slot, 1, j]], sem_s).wait()

    seed = pltpu.async_copy(
        table_hbm.at[pl.ds(pbase + t * ROWS_PER_TILE, ROWS_PER_TILE)],
        acc.at[pl.ds(t * ROWS_PER_TILE, ROWS_PER_TILE)], sem_g)
    stage(0, 0)
    gath(0)
    seed.wait()
    plsc.subcore_barrier()

    def body(i, carry):
      s0, s1 = carry

      @pl.when(i >= NSLOT - 1)
      def _():
        wait_scat(s1)

      @pl.when(i + 1 < NITER_AGG)
      def _():
        stage(i + 1, s1)
        gath(s1)

      wait_gath(s0)
      scat(s0)
      return s1, jnp.where(s1 == NSLOT - 1, 0, s1 + 1)

    lax.fori_loop(0, NITER_AGG, body,
                  (jnp.int32(0), jnp.int32(1)))
    for d in range(1, NSLOT):
      wait_scat((NITER_AGG - d) % NSLOT)
    plsc.subcore_barrier()
    pltpu.sync_copy(
        acc.at[pl.ds(t * ROWS_PER_TILE, ROWS_PER_TILE)],
        out_hbm.at[pl.ds(pbase + t * ROWS_PER_TILE, ROWS_PER_TILE)])
    plsc.subcore_barrier()


def _make_agg(nchunks):
  return pl.kernel(
      functools.partial(_agg_body, nchunks),
      out_type=jax.ShapeDtypeStruct((nchunks * NP, 32), jnp.float32),
      mesh=_mesh,
      compiler_params=_sc_params,
      scratch_types=[
          pltpu.VMEM_SHARED((NP, 32), jnp.float32),
          pltpu.VMEM((NSLOT, 2, K, CHUNK), jnp.int32),
          pltpu.VMEM((NSLOT, K, CHUNK, 32), jnp.float32),
          pltpu.SemaphoreType.DMA,
          pltpu.SemaphoreType.DMA,
      ],
  )


_agg2_call = _make_agg(2)
_agg4_call = _make_agg(4)


def _prep_body(degp_ref, x_ref, xp_ref, dinv_ref):
  d = degp_ref[0, :, 0:1] + degp_ref[1, :, 0:1] - 1.0
  d = jnp.maximum(d, 1.0)
  dinv = lax.rsqrt(d)
  dinv_ref[...] = jnp.broadcast_to(dinv, (RB, 16))
  xv = x_ref[...] * dinv
  xp_ref[0] = xv[:, :32]
  xp_ref[1] = jnp.concatenate(
      [xv[:, 32:F_IN], jnp.zeros((RB, 64 - F_IN), jnp.float32)], axis=1)


def _mid_body(agg1_ref, dinv_ref, w1_ref, w2_ref, b1_ref, zp_ref):
  dinv = dinv_ref[:, 0:1]
  a = jnp.concatenate([agg1_ref[0], agg1_ref[1]], axis=1) * dinv
  h = lax.dot_general(a, w1_ref[...], (((1,), (0,)), ((), ())),
                      precision=lax.Precision.HIGHEST,
                      preferred_element_type=jnp.float32) + b1_ref[...]
  h = jnp.where(h >= 0, h, 0.01 * h)
  z = lax.dot_general(h, w2_ref[...], (((1,), (0,)), ((), ())),
                      precision=lax.Precision.HIGHEST,
                      preferred_element_type=jnp.float32) * dinv
  for p in range(4):
    zp_ref[p] = z[:, 32 * p:32 * (p + 1)]


def _final_body(agg2_ref, dinv_ref, b2_ref, out_ref):
  dinv = dinv_ref[:, 0:1]
  z = jnp.concatenate([agg2_ref[p] for p in range(4)], axis=1)
  z = z * dinv + b2_ref[...]
  lane = lax.broadcasted_iota(jnp.int32, (RB, 128), 1)
  zm = jnp.where(lane >= OUT, -jnp.inf, z)
  m = jnp.max(zm, axis=1, keepdims=True)
  lse = jnp.log(jnp.sum(jnp.exp(zm - m), axis=1, keepdims=True)) + m
  out_ref[...] = (z - lse)[:, :OUT]


def kernel(x, edge_index, W1, b1, W2, b2):
  ones16 = jnp.ones((NP, 8), jnp.float32)
  xpad = jnp.zeros((NP, F_IN), jnp.float32).at[:N].set(x)

  padrow = N + jnp.arange(E_PAD - E, dtype=jnp.int32) % (NP - N)
  edge3 = jnp.concatenate(
      [edge_index, jnp.broadcast_to(padrow, (2, E_PAD - E))],
      axis=1).reshape(2, NCHUNK, CHUNK)

  degp = _deg_call(edge3, ones16)

  xp, dinv2d = pl.pallas_call(
      _prep_body,
      grid=(GRID,),
      in_specs=[
          pl.BlockSpec((2, RB, 8), lambda i: (0, i, 0)),
          pl.BlockSpec((RB, F_IN), lambda i: (i, 0)),
      ],
      out_specs=[
          pl.BlockSpec((2, RB, 32), lambda i: (0, i, 0)),
          pl.BlockSpec((RB, 16), lambda i: (i, 0)),
      ],
      out_shape=[
          jax.ShapeDtypeStruct((2, NP, 32), jnp.float32),
          jax.ShapeDtypeStruct((NP, 16), jnp.float32),
      ],
  )(degp, xpad)

  agg1 = _agg2_call(xp.reshape(2 * NP, 32), edge3)

  w1p = jnp.zeros((64, HID), jnp.float32).at[:F_IN].set(W1)
  w2p = jnp.zeros((HID, 128), jnp.float32).at[:, :OUT].set(W2)
  b2p = jnp.zeros((1, 128), jnp.float32).at[0, :OUT].set(b2)

  zp = pl.pallas_call(
      _mid_body,
      grid=(GRID,),
      in_specs=[
          pl.BlockSpec((2, RB, 32), lambda i: (0, i, 0)),
          pl.BlockSpec((RB, 16), lambda i: (i, 0)),
          pl.BlockSpec((64, HID), lambda i: (0, 0)),
          pl.BlockSpec((HID, 128), lambda i: (0, 0)),
          pl.BlockSpec((1, HID), lambda i: (0, 0)),
      ],
      out_specs=pl.BlockSpec((4, RB, 32), lambda i: (0, i, 0)),
      out_shape=jax.ShapeDtypeStruct((4, NP, 32), jnp.float32),
  )(agg1.reshape(2, NP, 32), dinv2d, w1p, w2p, b1.reshape(1, HID))

  agg2 = _agg4_call(zp.reshape(4 * NP, 32), edge3)

  out = pl.pallas_call(
      _final_body,
      grid=(GRID,),
      in_specs=[
          pl.BlockSpec((4, RB, 32), lambda i: (0, i, 0)),
          pl.BlockSpec((RB, 16), lambda i: (i, 0)),
          pl.BlockSpec((1, 128), lambda i: (0, 0)),
      ],
      out_specs=pl.BlockSpec((RB, OUT), lambda i: (i, 0)),
      out_shape=jax.ShapeDtypeStruct((N, OUT), jnp.float32),
  )(agg2.reshape(4, NP, 32), dinv2d, b2p)

  return out
